# baseline (speedup 1.0000x reference)
import jax
import jax.numpy as jnp
from jax.experimental import pallas as pl
from jax.experimental.pallas import tpu as pltpu

B = 1024
T = 128
OBS = 256
A = 128

_BB = 32


def _big_body(step_ref, obs_in, mask_in, obs_new, mask_new,
              obs_out, mask_out):
    s = step_ref[0]
    hit3 = jax.lax.broadcasted_iota(jnp.int32, (1, T, 1), 1) == s
    obs_out[...] = jnp.where(hit3, obs_new[...][:, None, :], obs_in[...])
    mask_out[...] = jnp.where(hit3, mask_new[...][:, None, :], mask_in[...])


def _small_body(step_ref, act_in, rew_in, lp_in, val_in,
                a_new, r_new, l_new, v_new,
                act_out, rew_out, lp_out, val_out):
    s = step_ref[0]
    hit2 = jax.lax.broadcasted_iota(jnp.int32, (1, T), 1) == s
    act_out[...] = jnp.where(hit2, a_new[...], act_in[...])
    rew_out[...] = jnp.where(hit2, r_new[...], rew_in[...])
    lp_out[...] = jnp.where(hit2, l_new[...], lp_in[...])
    hit2v = jax.lax.broadcasted_iota(jnp.int32, (1, T + 1), 1) == s
    val_out[...] = jnp.where(hit2v, v_new[...], val_in[...])


def kernel(state_obs, state_action_mask, state_actions, state_rewards,
           state_log_prob, state_values, state_advantages, state_targets,
           step, obs, action_mask, action, reward, log_prob, value):
    step_arr = jnp.asarray(step, jnp.int32).reshape((1,))

    new_obs, new_mask = pl.pallas_call(
        _big_body,
        grid=(B // _BB,),
        in_specs=[
            pl.BlockSpec(memory_space=pltpu.SMEM),
            pl.BlockSpec((_BB, T, OBS), lambda i: (i, 0, 0)),
            pl.BlockSpec((_BB, T, A), lambda i: (i, 0, 0)),
            pl.BlockSpec((_BB, OBS), lambda i: (i, 0)),
            pl.BlockSpec((_BB, A), lambda i: (i, 0)),
        ],
        out_specs=[
            pl.BlockSpec((_BB, T, OBS), lambda i: (i, 0, 0)),
            pl.BlockSpec((_BB, T, A), lambda i: (i, 0, 0)),
        ],
        out_shape=(
            jax.ShapeDtypeStruct((B, T, OBS), jnp.float32),
            jax.ShapeDtypeStruct((B, T, A), jnp.int8),
        ),
    )(step_arr, state_obs, state_action_mask.astype(jnp.int8),
      obs, action_mask.astype(jnp.int8))
    new_mask = new_mask.astype(jnp.bool_)

    full2 = lambda t_: pl.BlockSpec((B, t_), lambda: (0, 0))
    new_act, new_rew, new_lp, new_val = pl.pallas_call(
        _small_body,
        in_specs=[
            pl.BlockSpec(memory_space=pltpu.SMEM),
            full2(T), full2(T), full2(T), full2(T + 1),
            full2(1), full2(1), full2(1), full2(1),
        ],
        out_specs=[full2(T), full2(T), full2(T), full2(T + 1)],
        out_shape=(
            jax.ShapeDtypeStruct((B, T), jnp.int32),
            jax.ShapeDtypeStruct((B, T), jnp.float32),
            jax.ShapeDtypeStruct((B, T), jnp.float32),
            jax.ShapeDtypeStruct((B, T + 1), jnp.float32),
        ),
    )(step_arr, state_actions, state_rewards, state_log_prob, state_values,
      action.reshape(B, 1), reward.reshape(B, 1),
      log_prob.reshape(B, 1), value.reshape(B, 1))

    return (new_obs, new_mask, new_act, new_rew, new_lp, new_val,
            state_advantages, state_targets)


# speedup vs baseline: 1.0179x; 1.0179x over previous
import jax
import jax.numpy as jnp
from jax.experimental import pallas as pl
from jax.experimental.pallas import tpu as pltpu

B = 1024
T = 128
OBS = 256
A = 128

_BB = 64


def _body(step_ref,
          obs_in, mask_in, act_in, rew_in, lp_in, val_in,
          obs_new, mask_new, a_new, r_new, l_new, v_new,
          obs_out, mask_out, act_out, rew_out, lp_out, val_out):
    s = step_ref[0]
    hit3 = jax.lax.broadcasted_iota(jnp.int32, (1, T, 1), 1) == s
    obs_out[...] = jnp.where(hit3, obs_new[...][:, None, :], obs_in[...])
    mask_out[...] = jnp.where(hit3, mask_new[...][:, None, :], mask_in[...])
    hit2 = jax.lax.broadcasted_iota(jnp.int32, (1, T), 1) == s
    act_out[...] = jnp.where(hit2, a_new[...], act_in[...])
    rew_out[...] = jnp.where(hit2, r_new[...], rew_in[...])
    lp_out[...] = jnp.where(hit2, l_new[...], lp_in[...])
    hit2v = jax.lax.broadcasted_iota(jnp.int32, (1, T + 1), 1) == s
    val_out[...] = jnp.where(hit2v, v_new[...], val_in[...])


def kernel(state_obs, state_action_mask, state_actions, state_rewards,
           state_log_prob, state_values, state_advantages, state_targets,
           step, obs, action_mask, action, reward, log_prob, value):
    step_arr = jnp.asarray(step, jnp.int32).reshape((1,))

    slide3 = lambda t_, a_: pl.BlockSpec((_BB, t_, a_), lambda i: (i, 0, 0))
    slide2 = lambda t_: pl.BlockSpec((_BB, t_), lambda i: (i, 0))
    full2 = lambda t_: pl.BlockSpec((B, t_), lambda i: (0, 0))

    outs = pl.pallas_call(
        _body,
        grid=(B // _BB,),
        in_specs=[
            pl.BlockSpec(memory_space=pltpu.SMEM),
            slide3(T, OBS), slide3(T, A),
            full2(T), full2(T), full2(T), full2(T + 1),
            slide2(OBS), slide2(A),
            full2(1), full2(1), full2(1), full2(1),
        ],
        out_specs=[
            slide3(T, OBS), slide3(T, A),
            full2(T), full2(T), full2(T), full2(T + 1),
        ],
        out_shape=(
            jax.ShapeDtypeStruct((B, T, OBS), jnp.float32),
            jax.ShapeDtypeStruct((B, T, A), jnp.int8),
            jax.ShapeDtypeStruct((B, T), jnp.int32),
            jax.ShapeDtypeStruct((B, T), jnp.float32),
            jax.ShapeDtypeStruct((B, T), jnp.float32),
            jax.ShapeDtypeStruct((B, T + 1), jnp.float32),
        ),
    )(step_arr,
      state_obs, state_action_mask.astype(jnp.int8),
      state_actions, state_rewards, state_log_prob, state_values,
      obs, action_mask.astype(jnp.int8),
      action.reshape(B, 1), reward.reshape(B, 1),
      log_prob.reshape(B, 1), value.reshape(B, 1))

    new_obs, new_mask, new_act, new_rew, new_lp, new_val = outs
    return (new_obs, new_mask.astype(jnp.bool_), new_act, new_rew, new_lp,
            new_val, state_advantages, state_targets)
